# Initial kernel scaffold; baseline (speedup 1.0000x reference)
import jax
import jax.numpy as jnp
from jax.experimental import pallas as pl


def kernel(user_id, item_id, gender, profit_type, settle_cycle, item_catalog, item_tag, time_stamp):
    raise NotImplementedError("write your pallas kernel here")


# speedup vs baseline: 4.1040x; 4.1040x over previous
import functools

import jax
import jax.numpy as jnp
from jax import lax
from jax.experimental import pallas as pl
from jax.experimental.pallas import tpu as pltpu
from jax.experimental.pallas import tpu_sc as plsc

B = 4096
NUM_WORKERS = 32
ROWS_PER_W = B // NUM_WORKERS
CHUNK = 64
NGROUPS = CHUNK // 16
WIDTH = 1227

ITEM_BASE = 1
GENDER_BASE = 201
PROFIT_BASE = 205
SETTLE_BASE = 209
CATALOG_BASE = 213
USER_COL = 226
TAG_BASE = 227
NUM_TAGS = 20
USER_TOKENS = 12000

_BUF_WORDS = CHUNK * WIDTH
_ZERO_UNROLL = 12
_ZERO_ITERS = _BUF_WORDS // 16 // _ZERO_UNROLL


def _body(user_hbm, item_hbm, gender_hbm, profit_hbm, settle_hbm,
          catalog_hbm, tag_hbm, ts_hbm, out_hbm,
          ts_all, item_v, gender_v, profit_v, settle_v, catalog_v,
          user_v, tag_v, buf):
    wid = lax.axis_index("s") * 2 + lax.axis_index("c")
    base = wid * ROWS_PER_W

    pltpu.sync_copy(ts_hbm, ts_all)
    pltpu.sync_copy(item_hbm.at[pl.ds(base, ROWS_PER_W)], item_v)
    pltpu.sync_copy(gender_hbm.at[pl.ds(base, ROWS_PER_W)], gender_v)
    pltpu.sync_copy(profit_hbm.at[pl.ds(base, ROWS_PER_W)], profit_v)
    pltpu.sync_copy(settle_hbm.at[pl.ds(base, ROWS_PER_W)], settle_v)
    pltpu.sync_copy(catalog_hbm.at[pl.ds(base, ROWS_PER_W)], catalog_v)
    pltpu.sync_copy(user_hbm.at[pl.ds(base, ROWS_PER_W)], user_v)
    pltpu.sync_copy(tag_hbm.at[pl.ds(base, ROWS_PER_W)], tag_v)

    zeros16 = jnp.zeros((16,), jnp.float32)
    ones16 = jnp.ones((16,), jnp.float32)
    lanes = lax.iota(jnp.int32, 16)

    def lanesum(vec):
        s = vec[0]
        for i in range(1, 16):
            s = s + vec[i]
        return s

    def sum1(i, acc):
        return acc + ts_all[pl.ds(i * 16, 16)]
    mean = lanesum(lax.fori_loop(0, B // 16, sum1, zeros16)) * (1.0 / B)

    def sum2(i, acc):
        d = ts_all[pl.ds(i * 16, 16)] - mean
        return acc + d * d
    var = lanesum(lax.fori_loop(0, B // 16, sum2, zeros16)) * (1.0 / B)

    def zbody(i, carry):
        for k in range(_ZERO_UNROLL):
            buf[pl.ds((i * _ZERO_UNROLL + k) * 16, 16)] = zeros16
        return carry
    lax.fori_loop(0, _ZERO_ITERS, zbody, 0)

    v16 = ones16 * (var + 1e-6)
    iv = lax.bitcast_convert_type(v16, jnp.int32)
    y = lax.bitcast_convert_type(jnp.int32(0x5F3759DF) - (iv >> 1),
                                 jnp.float32)
    for _ in range(4):
        y = y * (1.5 - 0.5 * v16 * y * y)
    scale16 = y
    mean16 = ones16 * mean

    def scatter_chunk(c, val16, dense):
        def gbody(g, carry):
            off = c * CHUNK + g * 16
            rowbase = (g * 16 + lanes) * WIDTH
            item = item_v[pl.ds(off, 16)]
            plsc.store_scatter(buf, [rowbase + (item + ITEM_BASE)], val16)
            gen = gender_v[pl.ds(off, 16)]
            plsc.store_scatter(buf, [rowbase + (gen + GENDER_BASE)], val16)
            pro = profit_v[pl.ds(off, 16)]
            plsc.store_scatter(buf, [rowbase + (pro + PROFIT_BASE)], val16)
            stl = settle_v[pl.ds(off, 16)]
            plsc.store_scatter(buf, [rowbase + (stl + SETTLE_BASE)], val16)
            cat = catalog_v[pl.ds(off, 16)]
            plsc.store_scatter(buf, [rowbase + (cat + CATALOG_BASE)], val16)
            rows16 = off + lanes
            for j in range(NUM_TAGS):
                js = jnp.full((16,), j, jnp.int32)
                tg = plsc.load_gather(tag_v, [rows16, js])
                plsc.store_scatter(buf, [rowbase + (tg + TAG_BASE)], val16)
            if dense:
                tsv = ts_all[pl.ds(base + off, 16)]
                plsc.store_scatter(buf, [rowbase], (tsv - mean16) * scale16)
                u = user_v[pl.ds(off, 16)]
                uf = lax.rem(u, USER_TOKENS).astype(jnp.float32) / float(
                    USER_TOKENS)
                plsc.store_scatter(buf, [rowbase + USER_COL], uf)
            return carry
        lax.fori_loop(0, NGROUPS, gbody, 0)

    scatter_chunk(0, ones16, dense=True)
    pltpu.sync_copy(buf, out_hbm.at[pl.ds(base * WIDTH, _BUF_WORDS)])
    scatter_chunk(0, zeros16, dense=False)
    scatter_chunk(1, ones16, dense=True)
    pltpu.sync_copy(
        buf, out_hbm.at[pl.ds((base + CHUNK) * WIDTH, _BUF_WORDS)])


_preprocess_sc = pl.kernel(
    _body,
    out_type=jax.ShapeDtypeStruct((B * WIDTH,), jnp.float32),
    mesh=plsc.VectorSubcoreMesh(core_axis_name="c", subcore_axis_name="s"),
    compiler_params=pltpu.CompilerParams(needs_layout_passes=False),
    scratch_types=[
        pltpu.VMEM((B,), jnp.float32),
        pltpu.VMEM((ROWS_PER_W,), jnp.int32),
        pltpu.VMEM((ROWS_PER_W,), jnp.int32),
        pltpu.VMEM((ROWS_PER_W,), jnp.int32),
        pltpu.VMEM((ROWS_PER_W,), jnp.int32),
        pltpu.VMEM((ROWS_PER_W,), jnp.int32),
        pltpu.VMEM((ROWS_PER_W,), jnp.int32),
        pltpu.VMEM((ROWS_PER_W, NUM_TAGS), jnp.int32),
        pltpu.VMEM((_BUF_WORDS,), jnp.float32),
    ],
)


def kernel(user_id, item_id, gender, profit_type, settle_cycle,
           item_catalog, item_tag, time_stamp):
    out = _preprocess_sc(user_id, item_id, gender, profit_type,
                         settle_cycle, item_catalog, item_tag, time_stamp)
    return out.reshape(B, WIDTH)


# speedup vs baseline: 6.1553x; 1.4998x over previous
import jax
import jax.numpy as jnp
from jax import lax
from jax.experimental import pallas as pl
from jax.experimental.pallas import tpu as pltpu
from jax.experimental.pallas import tpu_sc as plsc

B = 4096
NUM_WORKERS = 32
ROWS_PER_W = B // NUM_WORKERS
CHUNK = 64
NGROUPS = CHUNK // 16
WIDTH = 1227

ITEM_BASE = 1
GENDER_BASE = 201
PROFIT_BASE = 205
SETTLE_BASE = 209
CATALOG_BASE = 213
USER_COL = 226
TAG_BASE = 227
NUM_TAGS = 20
USER_TOKENS = 12000

_FULL_STORES = WIDTH // 16
_TAIL_START = WIDTH - 16


def _body(user_hbm, item_hbm, gender_hbm, profit_hbm, settle_hbm,
          catalog_hbm, tag_hbm, ts_hbm, out_hbm,
          ts_all, item_v, gender_v, profit_v, settle_v, catalog_v,
          user_v, tag_v, buf):
    wid = lax.axis_index("s") * 2 + lax.axis_index("c")
    base = wid * ROWS_PER_W

    pltpu.sync_copy(ts_hbm, ts_all)
    pltpu.sync_copy(item_hbm.at[pl.ds(base, ROWS_PER_W)], item_v)
    pltpu.sync_copy(gender_hbm.at[pl.ds(base, ROWS_PER_W)], gender_v)
    pltpu.sync_copy(profit_hbm.at[pl.ds(base, ROWS_PER_W)], profit_v)
    pltpu.sync_copy(settle_hbm.at[pl.ds(base, ROWS_PER_W)], settle_v)
    pltpu.sync_copy(catalog_hbm.at[pl.ds(base, ROWS_PER_W)], catalog_v)
    pltpu.sync_copy(user_hbm.at[pl.ds(base, ROWS_PER_W)], user_v)
    pltpu.sync_copy(tag_hbm.at[pl.ds(base, ROWS_PER_W)], tag_v)

    zeros16 = jnp.zeros((16,), jnp.float32)
    ones16 = jnp.ones((16,), jnp.float32)
    lanes = lax.iota(jnp.int32, 16)

    def lanesum(vec):
        s = vec[0]
        for i in range(1, 16):
            s = s + vec[i]
        return s

    def sum1(i, acc):
        return acc + ts_all[pl.ds(i * 16, 16)]
    mean = lanesum(lax.fori_loop(0, B // 16, sum1, zeros16)) * (1.0 / B)

    def sum2(i, acc):
        d = ts_all[pl.ds(i * 16, 16)] - mean
        return acc + d * d
    var = lanesum(lax.fori_loop(0, B // 16, sum2, zeros16)) * (1.0 / B)

    def zbody(r, carry):
        for c in range(_FULL_STORES):
            buf[r, pl.ds(c * 16, 16)] = zeros16
        buf[r, pl.ds(_TAIL_START, 16)] = zeros16
        return carry
    lax.fori_loop(0, CHUNK, zbody, 0)

    v16 = ones16 * (var + 1e-6)
    iv = lax.bitcast_convert_type(v16, jnp.int32)
    y = lax.bitcast_convert_type(jnp.int32(0x5F3759DF) - (iv >> 1),
                                 jnp.float32)
    for _ in range(4):
        y = y * (1.5 - 0.5 * v16 * y * y)
    scale16 = y
    mean16 = ones16 * mean

    def scatter_chunk(c, val16, dense):
        def gbody(g, carry):
            off = c * CHUNK + g * 16
            rows16 = g * 16 + lanes
            item = item_v[pl.ds(off, 16)]
            plsc.store_scatter(buf, [rows16, item + ITEM_BASE], val16)
            gen = gender_v[pl.ds(off, 16)]
            plsc.store_scatter(buf, [rows16, gen + GENDER_BASE], val16)
            pro = profit_v[pl.ds(off, 16)]
            plsc.store_scatter(buf, [rows16, pro + PROFIT_BASE], val16)
            stl = settle_v[pl.ds(off, 16)]
            plsc.store_scatter(buf, [rows16, stl + SETTLE_BASE], val16)
            cat = catalog_v[pl.ds(off, 16)]
            plsc.store_scatter(buf, [rows16, cat + CATALOG_BASE], val16)
            in_rows16 = off + lanes
            for j in range(NUM_TAGS):
                js = jnp.full((16,), j, jnp.int32)
                tg = plsc.load_gather(tag_v, [in_rows16, js])
                plsc.store_scatter(buf, [rows16, tg + TAG_BASE], val16)
            if dense:
                tsv = ts_all[pl.ds(base + off, 16)]
                zc = jnp.zeros((16,), jnp.int32)
                plsc.store_scatter(buf, [rows16, zc],
                                   (tsv - mean16) * scale16)
                u = user_v[pl.ds(off, 16)]
                uf = lax.rem(u, USER_TOKENS).astype(jnp.float32) * (
                    1.0 / USER_TOKENS)
                plsc.store_scatter(buf, [rows16, zc + USER_COL], uf)
            return carry
        lax.fori_loop(0, NGROUPS, gbody, 0)

    scatter_chunk(0, ones16, dense=True)
    pltpu.sync_copy(buf, out_hbm.at[pl.ds(base, CHUNK)])
    scatter_chunk(0, zeros16, dense=False)
    scatter_chunk(1, ones16, dense=True)
    pltpu.sync_copy(buf, out_hbm.at[pl.ds(base + CHUNK, CHUNK)])


_preprocess_sc = pl.kernel(
    _body,
    out_type=jax.ShapeDtypeStruct((B, WIDTH), jnp.float32),
    mesh=plsc.VectorSubcoreMesh(core_axis_name="c", subcore_axis_name="s"),
    compiler_params=pltpu.CompilerParams(needs_layout_passes=False),
    scratch_types=[
        pltpu.VMEM((B,), jnp.float32),
        pltpu.VMEM((ROWS_PER_W,), jnp.int32),
        pltpu.VMEM((ROWS_PER_W,), jnp.int32),
        pltpu.VMEM((ROWS_PER_W,), jnp.int32),
        pltpu.VMEM((ROWS_PER_W,), jnp.int32),
        pltpu.VMEM((ROWS_PER_W,), jnp.int32),
        pltpu.VMEM((ROWS_PER_W,), jnp.int32),
        pltpu.VMEM((ROWS_PER_W, NUM_TAGS), jnp.int32),
        pltpu.VMEM((CHUNK, WIDTH), jnp.float32),
    ],
)


def kernel(user_id, item_id, gender, profit_type, settle_cycle,
           item_catalog, item_tag, time_stamp):
    return _preprocess_sc(user_id, item_id, gender, profit_type,
                          settle_cycle, item_catalog, item_tag, time_stamp)


# speedup vs baseline: 6.9325x; 1.1263x over previous
import jax
import jax.numpy as jnp
from jax import lax
from jax.experimental import pallas as pl
from jax.experimental.pallas import tpu as pltpu
from jax.experimental.pallas import tpu_sc as plsc

B = 4096
NUM_WORKERS = 32
ROWS_PER_W = B // NUM_WORKERS
CHUNK = 32
NCHUNKS = ROWS_PER_W // CHUNK
NGROUPS = CHUNK // 16
WIDTH = 1227

ITEM_BASE = 1
GENDER_BASE = 201
PROFIT_BASE = 205
SETTLE_BASE = 209
CATALOG_BASE = 213
USER_COL = 226
TAG_BASE = 227
NUM_TAGS = 20
USER_TOKENS = 12000

_FULL_STORES = WIDTH // 16
_TAIL_START = WIDTH - 16


def _body(user_hbm, item_hbm, gender_hbm, profit_hbm, settle_hbm,
          catalog_hbm, tag_hbm, ts_hbm, out_hbm,
          ts_all, item_v, gender_v, profit_v, settle_v, catalog_v,
          user_v, tag_v, buf0, buf1, sem_in, sem0, sem1):
    wid = lax.axis_index("s") * 2 + lax.axis_index("c")
    base = wid * ROWS_PER_W

    in_copies = [
        pltpu.async_copy(ts_hbm, ts_all, sem_in),
        pltpu.async_copy(item_hbm.at[pl.ds(base, ROWS_PER_W)], item_v,
                         sem_in),
        pltpu.async_copy(gender_hbm.at[pl.ds(base, ROWS_PER_W)], gender_v,
                         sem_in),
        pltpu.async_copy(profit_hbm.at[pl.ds(base, ROWS_PER_W)], profit_v,
                         sem_in),
        pltpu.async_copy(settle_hbm.at[pl.ds(base, ROWS_PER_W)], settle_v,
                         sem_in),
        pltpu.async_copy(catalog_hbm.at[pl.ds(base, ROWS_PER_W)],
                         catalog_v, sem_in),
        pltpu.async_copy(user_hbm.at[pl.ds(base, ROWS_PER_W)], user_v,
                         sem_in),
        pltpu.async_copy(tag_hbm.at[pl.ds(base, ROWS_PER_W)], tag_v,
                         sem_in),
    ]

    zeros16 = jnp.zeros((16,), jnp.float32)
    ones16 = jnp.ones((16,), jnp.float32)
    lanes = lax.iota(jnp.int32, 16)

    for buf in (buf0, buf1):
        def zbody(r, carry, buf=buf):
            for c in range(_FULL_STORES):
                buf[r, pl.ds(c * 16, 16)] = zeros16
            buf[r, pl.ds(_TAIL_START, 16)] = zeros16
            return carry
        lax.fori_loop(0, CHUNK, zbody, 0)

    for cp in in_copies:
        cp.wait()

    def lanesum(vec):
        s = vec[0]
        for i in range(1, 16):
            s = s + vec[i]
        return s

    def sum1(i, acc):
        return acc + ts_all[pl.ds(i * 16, 16)]
    mean = lanesum(lax.fori_loop(0, B // 16, sum1, zeros16)) * (1.0 / B)

    def sum2(i, acc):
        d = ts_all[pl.ds(i * 16, 16)] - mean
        return acc + d * d
    var = lanesum(lax.fori_loop(0, B // 16, sum2, zeros16)) * (1.0 / B)

    v16 = ones16 * (var + 1e-6)
    iv = lax.bitcast_convert_type(v16, jnp.int32)
    y = lax.bitcast_convert_type(jnp.int32(0x5F3759DF) - (iv >> 1),
                                 jnp.float32)
    for _ in range(4):
        y = y * (1.5 - 0.5 * v16 * y * y)
    scale16 = y
    mean16 = ones16 * mean

    def scatter_chunk(c, buf, val16, dense):
        def gbody(g, carry):
            off = c * CHUNK + g * 16
            rows16 = g * 16 + lanes
            item = item_v[pl.ds(off, 16)]
            plsc.store_scatter(buf, [rows16, item + ITEM_BASE], val16)
            gen = gender_v[pl.ds(off, 16)]
            plsc.store_scatter(buf, [rows16, gen + GENDER_BASE], val16)
            pro = profit_v[pl.ds(off, 16)]
            plsc.store_scatter(buf, [rows16, pro + PROFIT_BASE], val16)
            stl = settle_v[pl.ds(off, 16)]
            plsc.store_scatter(buf, [rows16, stl + SETTLE_BASE], val16)
            cat = catalog_v[pl.ds(off, 16)]
            plsc.store_scatter(buf, [rows16, cat + CATALOG_BASE], val16)
            in_rows16 = off + lanes
            for j in range(NUM_TAGS):
                js = jnp.full((16,), j, jnp.int32)
                tg = plsc.load_gather(tag_v, [in_rows16, js])
                plsc.store_scatter(buf, [rows16, tg + TAG_BASE], val16)
            if dense:
                tsv = ts_all[pl.ds(base + off, 16)]
                zc = jnp.zeros((16,), jnp.int32)
                plsc.store_scatter(buf, [rows16, zc],
                                   (tsv - mean16) * scale16)
                u = user_v[pl.ds(off, 16)]
                uf = lax.rem(u, USER_TOKENS).astype(jnp.float32) * (
                    1.0 / USER_TOKENS)
                plsc.store_scatter(buf, [rows16, zc + USER_COL], uf)
            return carry
        lax.fori_loop(0, NGROUPS, gbody, 0)

    bufs = (buf0, buf1)
    sems = (sem0, sem1)
    out_dma = [None] * NCHUNKS
    for c in range(NCHUNKS):
        buf = bufs[c % 2]
        if c >= 2:
            out_dma[c - 2].wait()
            scatter_chunk(c - 2, buf, zeros16, dense=False)
        scatter_chunk(c, buf, ones16, dense=True)
        out_dma[c] = pltpu.async_copy(
            buf, out_hbm.at[pl.ds(base + c * CHUNK, CHUNK)], sems[c % 2])
    out_dma[NCHUNKS - 2].wait()
    out_dma[NCHUNKS - 1].wait()


_preprocess_sc = pl.kernel(
    _body,
    out_type=jax.ShapeDtypeStruct((B, WIDTH), jnp.float32),
    mesh=plsc.VectorSubcoreMesh(core_axis_name="c", subcore_axis_name="s"),
    compiler_params=pltpu.CompilerParams(needs_layout_passes=False),
    scratch_types=[
        pltpu.VMEM((B,), jnp.float32),
        pltpu.VMEM((ROWS_PER_W,), jnp.int32),
        pltpu.VMEM((ROWS_PER_W,), jnp.int32),
        pltpu.VMEM((ROWS_PER_W,), jnp.int32),
        pltpu.VMEM((ROWS_PER_W,), jnp.int32),
        pltpu.VMEM((ROWS_PER_W,), jnp.int32),
        pltpu.VMEM((ROWS_PER_W,), jnp.int32),
        pltpu.VMEM((ROWS_PER_W, NUM_TAGS), jnp.int32),
        pltpu.VMEM((CHUNK, WIDTH), jnp.float32),
        pltpu.VMEM((CHUNK, WIDTH), jnp.float32),
        pltpu.SemaphoreType.DMA,
        pltpu.SemaphoreType.DMA,
        pltpu.SemaphoreType.DMA,
    ],
)


def kernel(user_id, item_id, gender, profit_type, settle_cycle,
           item_catalog, item_tag, time_stamp):
    return _preprocess_sc(user_id, item_id, gender, profit_type,
                          settle_cycle, item_catalog, item_tag, time_stamp)
